# baseline (speedup 1.0000x reference)
import functools

import jax
import jax.numpy as jnp
from jax import lax
from jax.experimental import pallas as pl
from jax.experimental.pallas import tpu as pltpu
from jax.experimental.pallas import tpu_sc as plsc

_PATCH = 16
_RESIZE = 512
_ORIG_W, _ORIG_H = 600, 900

_NW = 32


def _sc_body(vpw, grid_n,
             xs_h, ys_h, zs_h, coef_h, out_h,
             x_v, y_v, z_v, coef_v, idx_v):
    wid = lax.axis_index("s") * 2 + lax.axis_index("c")
    base = wid * vpw
    pltpu.sync_copy(xs_h.at[pl.ds(base, vpw)], x_v)
    pltpu.sync_copy(ys_h.at[pl.ds(base, vpw)], y_v)
    pltpu.sync_copy(zs_h.at[pl.ds(base, vpw)], z_v)
    pltpu.sync_copy(coef_h, coef_v)
    c = [coef_v[i] for i in range(23)]

    for i in range(vpw // 16):
        sl = pl.ds(i * 16, 16)
        x = x_v[sl]
        y = y_v[sl]
        z = z_v[sl]
        cam0 = c[0] * x + c[1] * y + c[2] * z + c[3]
        cam1 = c[4] * x + c[5] * y + c[6] * z + c[7]
        cam2 = c[8] * x + c[9] * y + c[10] * z + c[11]
        p0 = c[12] * cam0 + c[13] * cam1 + c[14] * cam2
        p1 = c[15] * cam0 + c[16] * cam1 + c[17] * cam2
        p2 = c[18] * cam0 + c[19] * cam1 + c[20] * cam2
        den = p2 + 1e-6
        uf = jnp.clip((p0 / den) * c[21] * (1.0 / _PATCH), -1e9, 1e9)
        vf = jnp.clip((p1 / den) * c[22] * (1.0 / _PATCH), -1e9, 1e9)
        px = jnp.clip(uf.astype(jnp.int32), 0, grid_n - 1)
        py = jnp.clip(vf.astype(jnp.int32), 0, grid_n - 1)
        idx_v[sl] = px * grid_n + py

    pltpu.sync_copy(idx_v, out_h.at[pl.ds(base, vpw)])


def _sc_index(xs, ys, zs, coef, grid_n):
    BV = xs.shape[0]
    vpw = BV // _NW
    mesh = plsc.VectorSubcoreMesh(core_axis_name="c", subcore_axis_name="s")
    fn = pl.kernel(
        functools.partial(_sc_body, vpw, grid_n),
        out_type=jax.ShapeDtypeStruct((BV,), jnp.int32),
        mesh=mesh,
        scratch_types=[
            pltpu.VMEM((vpw,), jnp.float32),
            pltpu.VMEM((vpw,), jnp.float32),
            pltpu.VMEM((vpw,), jnp.float32),
            pltpu.VMEM(coef.shape, jnp.float32),
            pltpu.VMEM((vpw,), jnp.int32),
        ],
    )
    return fn(xs, ys, zs, coef)


def _mlp_body(sub, idx_ref, pt_ref, vf_ref, w1b_ref, w1a_ref, b1_ref, w2_ref,
              b2_ref, out_ref):
    M = pt_ref.shape[2]
    acc = pt_ref[0, 0]
    for v in range(1, pt_ref.shape[1]):
        acc = acc + pt_ref[0, v]
    mean = acc / float(pt_ref.shape[1])
    tab = jnp.dot(mean, w1b_ref[:],
                  preferred_element_type=jnp.float32).astype(jnp.bfloat16)
    taba = jnp.concatenate([tab, w1a_ref[:].astype(jnp.bfloat16)], axis=0)
    w2 = w2_ref[:].astype(jnp.bfloat16)
    for k in range(vf_ref.shape[1] // sub):
        sl = pl.ds(k * sub, sub)
        idxb = idx_ref[0, 0, sl]
        oh = (idxb[:, None] == lax.broadcasted_iota(
            jnp.int32, (sub, M), 1)).astype(jnp.bfloat16)
        ohv = jnp.concatenate([oh, vf_ref[0, sl].astype(jnp.bfloat16)],
                              axis=1)
        ga = jnp.dot(ohv, taba, preferred_element_type=jnp.float32)
        h = jnp.maximum(ga + b1_ref[:], 0.0).astype(jnp.bfloat16)
        out_ref[0, sl] = (jnp.dot(h, w2, preferred_element_type=jnp.float32)
                          + b2_ref[:])


def _mlp(idx3, patch_tokens, vf, w1b, w1a, b1, w2, b2):
    B, NV, M, D = patch_tokens.shape
    V = vf.shape[1]
    PF = vf.shape[2]
    H = w1a.shape[1]
    O = w2.shape[1]
    sub = 2048
    return pl.pallas_call(
        functools.partial(_mlp_body, sub),
        grid=(B,),
        in_specs=[
            pl.BlockSpec((1, 1, V), lambda b: (b, 0, 0)),
            pl.BlockSpec((1, NV, M, D), lambda b: (b, 0, 0, 0)),
            pl.BlockSpec((1, V, PF), lambda b: (b, 0, 0)),
            pl.BlockSpec((D, H), lambda b: (0, 0)),
            pl.BlockSpec((PF, H), lambda b: (0, 0)),
            pl.BlockSpec((1, H), lambda b: (0, 0)),
            pl.BlockSpec((H, O), lambda b: (0, 0)),
            pl.BlockSpec((1, O), lambda b: (0, 0)),
        ],
        out_specs=pl.BlockSpec((1, V, O), lambda b: (b, 0, 0)),
        out_shape=jax.ShapeDtypeStruct((B, V, O), jnp.float32),
    )(idx3, patch_tokens, vf, w1b, w1a, b1, w2, b2)


def kernel(patch_tokens, voxel_features, voxel_coords, K, Rt, W1, b1, W2, b2):
    B, NV, M, D = patch_tokens.shape
    _, V, PF = voxel_features.shape
    H = W1.shape[1]
    O = W2.shape[1]
    BV = B * V
    grid_n = _RESIZE // _PATCH

    w1a = W1[:PF]
    w1b = W1[PF:]

    coords = voxel_coords.reshape(BV, 3)
    xs = coords[:, 0]
    ys = coords[:, 1]
    zs = coords[:, 2]
    scale = jnp.asarray([_RESIZE / _ORIG_W, _RESIZE / _ORIG_H],
                        dtype=jnp.float32)
    vals = jnp.concatenate([
        Rt.reshape(-1), K.reshape(-1), scale,
        jnp.zeros((1,), jnp.float32)])
    coef = jnp.broadcast_to(vals[:, None], (24, 16))

    idx = _sc_index(xs, ys, zs, coef, grid_n)
    idx3 = idx.reshape(B, 1, V)

    out = _mlp(idx3, patch_tokens, voxel_features, w1b, w1a,
               b1.reshape(1, H), W2, b2.reshape(1, O))
    return out


# speedup vs baseline: 1.0830x; 1.0830x over previous
import functools

import jax
import jax.numpy as jnp
from jax import lax
from jax.experimental import pallas as pl
from jax.experimental.pallas import tpu as pltpu
from jax.experimental.pallas import tpu_sc as plsc

_PATCH = 16
_RESIZE = 512
_ORIG_W, _ORIG_H = 600, 900

_NW = 32


def _sc_body(vpw, grid_n,
             xs_h, ys_h, zs_h, coef_h, out_h,
             x_v, y_v, z_v, coef_v, idx_v):
    wid = lax.axis_index("s") * 2 + lax.axis_index("c")
    base = wid * vpw
    pltpu.sync_copy(xs_h.at[pl.ds(base, vpw)], x_v)
    pltpu.sync_copy(ys_h.at[pl.ds(base, vpw)], y_v)
    pltpu.sync_copy(zs_h.at[pl.ds(base, vpw)], z_v)
    pltpu.sync_copy(coef_h, coef_v)
    c = [coef_v[i] for i in range(23)]

    for i in range(vpw // 16):
        sl = pl.ds(i * 16, 16)
        x = x_v[sl]
        y = y_v[sl]
        z = z_v[sl]
        cam0 = c[0] * x + c[1] * y + c[2] * z + c[3]
        cam1 = c[4] * x + c[5] * y + c[6] * z + c[7]
        cam2 = c[8] * x + c[9] * y + c[10] * z + c[11]
        p0 = c[12] * cam0 + c[13] * cam1 + c[14] * cam2
        p1 = c[15] * cam0 + c[16] * cam1 + c[17] * cam2
        p2 = c[18] * cam0 + c[19] * cam1 + c[20] * cam2
        den = p2 + 1e-6
        uf = jnp.clip((p0 / den) * c[21] * (1.0 / _PATCH), -1e9, 1e9)
        vf = jnp.clip((p1 / den) * c[22] * (1.0 / _PATCH), -1e9, 1e9)
        px = jnp.clip(uf.astype(jnp.int32), 0, grid_n - 1)
        py = jnp.clip(vf.astype(jnp.int32), 0, grid_n - 1)
        idx_v[sl] = px * grid_n + py

    pltpu.sync_copy(idx_v, out_h.at[pl.ds(base, vpw)])


def _sc_index(xs, ys, zs, coef, grid_n):
    BV = xs.shape[0]
    vpw = BV // _NW
    mesh = plsc.VectorSubcoreMesh(core_axis_name="c", subcore_axis_name="s")
    fn = pl.kernel(
        functools.partial(_sc_body, vpw, grid_n),
        out_type=jax.ShapeDtypeStruct((BV,), jnp.int32),
        mesh=mesh,
        scratch_types=[
            pltpu.VMEM((vpw,), jnp.float32),
            pltpu.VMEM((vpw,), jnp.float32),
            pltpu.VMEM((vpw,), jnp.float32),
            pltpu.VMEM(coef.shape, jnp.float32),
            pltpu.VMEM((vpw,), jnp.int32),
        ],
    )
    return fn(xs, ys, zs, coef)


def _mlp_body(sub, idx_ref, pt_ref, vf_ref, w1b_ref, w1a_ref, b1_ref, w2_ref,
              b2_ref, out_ref):
    M = pt_ref.shape[2]
    acc = pt_ref[0, 0]
    for v in range(1, pt_ref.shape[1]):
        acc = acc + pt_ref[0, v]
    mean = acc / float(pt_ref.shape[1])
    tab = jnp.dot(mean, w1b_ref[:],
                  preferred_element_type=jnp.float32).astype(jnp.bfloat16)
    w2 = w2_ref[:].astype(jnp.bfloat16)
    for k in range(vf_ref.shape[1] // sub):
        sl = pl.ds(k * sub, sub)
        idxb = idx_ref[0, 0, sl]
        oh = (idxb[:, None] == lax.broadcasted_iota(
            jnp.int32, (sub, M), 1)).astype(jnp.bfloat16)
        g = jnp.dot(oh, tab, preferred_element_type=jnp.float32)
        a = jnp.dot(vf_ref[0, sl], w1a_ref[:],
                    preferred_element_type=jnp.float32)
        h = jnp.maximum(a + g + b1_ref[:], 0.0).astype(jnp.bfloat16)
        out_ref[0, sl] = (jnp.dot(h, w2, preferred_element_type=jnp.float32)
                          + b2_ref[:])


def _mlp(idx3, patch_tokens, vf, w1b, w1a, b1, w2, b2):
    B, NV, M, D = patch_tokens.shape
    V = vf.shape[1]
    PF = vf.shape[2]
    H = w1a.shape[1]
    O = w2.shape[1]
    sub = 2048
    return pl.pallas_call(
        functools.partial(_mlp_body, sub),
        grid=(B,),
        in_specs=[
            pl.BlockSpec((1, 1, V), lambda b: (b, 0, 0)),
            pl.BlockSpec((1, NV, M, D), lambda b: (b, 0, 0, 0)),
            pl.BlockSpec((1, V, PF), lambda b: (b, 0, 0)),
            pl.BlockSpec((D, H), lambda b: (0, 0)),
            pl.BlockSpec((PF, H), lambda b: (0, 0)),
            pl.BlockSpec((1, H), lambda b: (0, 0)),
            pl.BlockSpec((H, O), lambda b: (0, 0)),
            pl.BlockSpec((1, O), lambda b: (0, 0)),
        ],
        out_specs=pl.BlockSpec((1, V, O), lambda b: (b, 0, 0)),
        out_shape=jax.ShapeDtypeStruct((B, V, O), jnp.float32),
    )(idx3, patch_tokens, vf, w1b, w1a, b1, w2, b2)


def kernel(patch_tokens, voxel_features, voxel_coords, K, Rt, W1, b1, W2, b2):
    B, NV, M, D = patch_tokens.shape
    _, V, PF = voxel_features.shape
    H = W1.shape[1]
    O = W2.shape[1]
    BV = B * V
    grid_n = _RESIZE // _PATCH

    w1a = W1[:PF]
    w1b = W1[PF:]

    coords = voxel_coords.reshape(BV, 3)
    xs = coords[:, 0]
    ys = coords[:, 1]
    zs = coords[:, 2]
    scale = jnp.asarray([_RESIZE / _ORIG_W, _RESIZE / _ORIG_H],
                        dtype=jnp.float32)
    vals = jnp.concatenate([
        Rt.reshape(-1), K.reshape(-1), scale,
        jnp.zeros((1,), jnp.float32)])
    coef = jnp.broadcast_to(vals[:, None], (24, 16))

    idx = _sc_index(xs, ys, zs, coef, grid_n)
    idx3 = idx.reshape(B, 1, V)

    out = _mlp(idx3, patch_tokens, voxel_features, w1b, w1a,
               b1.reshape(1, H), W2, b2.reshape(1, O))
    return out


# speedup vs baseline: 1.0954x; 1.0115x over previous
import functools

import jax
import jax.numpy as jnp
from jax import lax
from jax.experimental import pallas as pl
from jax.experimental.pallas import tpu as pltpu
from jax.experimental.pallas import tpu_sc as plsc

_PATCH = 16
_RESIZE = 512
_ORIG_W, _ORIG_H = 600, 900

_NW = 32


def _sc_body(vpw, grid_n,
             xs_h, ys_h, zs_h, coef_h, out_h,
             x_v, y_v, z_v, coef_v, idx_v):
    wid = lax.axis_index("s") * 2 + lax.axis_index("c")
    base = wid * vpw
    pltpu.sync_copy(xs_h.at[pl.ds(base, vpw)], x_v)
    pltpu.sync_copy(ys_h.at[pl.ds(base, vpw)], y_v)
    pltpu.sync_copy(zs_h.at[pl.ds(base, vpw)], z_v)
    pltpu.sync_copy(coef_h, coef_v)
    c = [coef_v[i] for i in range(23)]

    for i in range(vpw // 16):
        sl = pl.ds(i * 16, 16)
        x = x_v[sl]
        y = y_v[sl]
        z = z_v[sl]
        cam0 = c[0] * x + c[1] * y + c[2] * z + c[3]
        cam1 = c[4] * x + c[5] * y + c[6] * z + c[7]
        cam2 = c[8] * x + c[9] * y + c[10] * z + c[11]
        p0 = c[12] * cam0 + c[13] * cam1 + c[14] * cam2
        p1 = c[15] * cam0 + c[16] * cam1 + c[17] * cam2
        p2 = c[18] * cam0 + c[19] * cam1 + c[20] * cam2
        den = p2 + 1e-6
        uf = jnp.clip((p0 / den) * c[21] * (1.0 / _PATCH), -1e9, 1e9)
        vf = jnp.clip((p1 / den) * c[22] * (1.0 / _PATCH), -1e9, 1e9)
        px = jnp.clip(uf.astype(jnp.int32), 0, grid_n - 1)
        py = jnp.clip(vf.astype(jnp.int32), 0, grid_n - 1)
        idx_v[sl] = px * grid_n + py

    pltpu.sync_copy(idx_v, out_h.at[pl.ds(base, vpw)])


def _sc_index(xs, ys, zs, coef, grid_n):
    BV = xs.shape[0]
    vpw = BV // _NW
    mesh = plsc.VectorSubcoreMesh(core_axis_name="c", subcore_axis_name="s")
    fn = pl.kernel(
        functools.partial(_sc_body, vpw, grid_n),
        out_type=jax.ShapeDtypeStruct((BV,), jnp.int32),
        mesh=mesh,
        scratch_types=[
            pltpu.VMEM((vpw,), jnp.float32),
            pltpu.VMEM((vpw,), jnp.float32),
            pltpu.VMEM((vpw,), jnp.float32),
            pltpu.VMEM(coef.shape, jnp.float32),
            pltpu.VMEM((vpw,), jnp.int32),
        ],
    )
    return fn(xs, ys, zs, coef)


def _mlp_body(sub, idx_ref, pt_ref, vf_ref, w1b_ref, w1a_ref, b1_ref, w2_ref,
              b2_ref, out_ref):
    M = pt_ref.shape[2]
    acc = pt_ref[0, 0]
    for v in range(1, pt_ref.shape[1]):
        acc = acc + pt_ref[0, v]
    mean = acc / float(pt_ref.shape[1])
    tab = jnp.dot(mean, w1b_ref[:],
                  preferred_element_type=jnp.float32).astype(jnp.bfloat16)
    w2 = w2_ref[:].astype(jnp.bfloat16)
    for k in range(vf_ref.shape[1] // sub):
        sl = pl.ds(k * sub, sub)
        idxb = idx_ref[0, 0, sl]
        oh = (idxb[:, None] == lax.broadcasted_iota(
            jnp.int32, (sub, M), 1)).astype(jnp.bfloat16)
        g = jnp.dot(oh, tab, preferred_element_type=jnp.float32)
        a = jnp.dot(vf_ref[0, sl], w1a_ref[:],
                    preferred_element_type=jnp.float32)
        h = jnp.maximum(a + g + b1_ref[:], 0.0).astype(jnp.bfloat16)
        out_ref[0, sl] = (jnp.dot(h, w2, preferred_element_type=jnp.float32)
                          + b2_ref[:])


def _mlp(idx3, patch_tokens, vf, w1b, w1a, b1, w2, b2):
    B, NV, M, D = patch_tokens.shape
    V = vf.shape[1]
    PF = vf.shape[2]
    H = w1a.shape[1]
    O = w2.shape[1]
    sub = 4096
    return pl.pallas_call(
        functools.partial(_mlp_body, sub),
        grid=(B,),
        in_specs=[
            pl.BlockSpec((1, 1, V), lambda b: (b, 0, 0)),
            pl.BlockSpec((1, NV, M, D), lambda b: (b, 0, 0, 0)),
            pl.BlockSpec((1, V, PF), lambda b: (b, 0, 0)),
            pl.BlockSpec((D, H), lambda b: (0, 0)),
            pl.BlockSpec((PF, H), lambda b: (0, 0)),
            pl.BlockSpec((1, H), lambda b: (0, 0)),
            pl.BlockSpec((H, O), lambda b: (0, 0)),
            pl.BlockSpec((1, O), lambda b: (0, 0)),
        ],
        out_specs=pl.BlockSpec((1, V, O), lambda b: (b, 0, 0)),
        out_shape=jax.ShapeDtypeStruct((B, V, O), jnp.float32),
    )(idx3, patch_tokens, vf, w1b, w1a, b1, w2, b2)


def kernel(patch_tokens, voxel_features, voxel_coords, K, Rt, W1, b1, W2, b2):
    B, NV, M, D = patch_tokens.shape
    _, V, PF = voxel_features.shape
    H = W1.shape[1]
    O = W2.shape[1]
    BV = B * V
    grid_n = _RESIZE // _PATCH

    w1a = W1[:PF]
    w1b = W1[PF:]

    coords = voxel_coords.reshape(BV, 3)
    xs = coords[:, 0]
    ys = coords[:, 1]
    zs = coords[:, 2]
    scale = jnp.asarray([_RESIZE / _ORIG_W, _RESIZE / _ORIG_H],
                        dtype=jnp.float32)
    vals = jnp.concatenate([
        Rt.reshape(-1), K.reshape(-1), scale,
        jnp.zeros((1,), jnp.float32)])
    coef = jnp.broadcast_to(vals[:, None], (24, 16))

    idx = _sc_index(xs, ys, zs, coef, grid_n)
    idx3 = idx.reshape(B, 1, V)

    out = _mlp(idx3, patch_tokens, voxel_features, w1b, w1a,
               b1.reshape(1, H), W2, b2.reshape(1, O))
    return out
